# baseline (speedup 1.0000x reference)
import functools

import jax
import jax.numpy as jnp
from jax import lax
from jax.experimental import pallas as pl
from jax.experimental.pallas import tpu as pltpu
from jax.experimental.pallas import tpu_sc as plsc

N = 4096
K = 20
RB = 256
RBM = 256
NW = 32
GRR = 64
WIN = 1280


def _select_topk(work, iot, idx_ref):
    BIG = jnp.int32(2 ** 30)
    cols = []
    for _ in range(K):
        m = jnp.min(work, axis=1, keepdims=True)
        cand = jnp.where(work == m, iot, BIG)
        im = jnp.min(cand, axis=1, keepdims=True)
        cols.append(im)
        sel = iot == im
        work = jnp.where(sel, jnp.inf, work)
        iot = jnp.where(sel, BIG, iot)
    idx_ref[...] = jnp.concatenate(cols, axis=1)


def _knn_body(c0_ref, ok_ref, x_ref, xT_ref, x16_ref, xT16_ref, brow_ref,
              bcol_ref, idx_ref):
    i = pl.program_id(0)
    xb = x_ref[...]
    sqr = jnp.sum(xb * xb, axis=1, keepdims=True)
    brow1 = brow_ref[...][:, :1]
    c0 = pl.multiple_of(c0_ref[i], 128)
    ok = ok_ref[i] == 1

    @pl.when(ok)
    def _():
        xTf = jnp.concatenate(
            [xT_ref[:, :128], xT_ref[:, pl.ds(c0, WIN)]], axis=1)
        xTc = jnp.concatenate(
            [xT16_ref[:, :128], xT16_ref[:, pl.ds(c0, WIN)]], axis=1)
        bcolc = jnp.concatenate(
            [bcol_ref[:, :128], bcol_ref[:, pl.ds(c0, WIN)]], axis=1)[:1, :]
        sqc = jnp.sum(xTf * xTf, axis=0, keepdims=True)
        G = jnp.dot(x16_ref[...], xTc, preferred_element_type=jnp.float32)
        d2 = (sqr + sqc) - 2.0 * G
        work = jnp.where(brow1 != bcolc, jnp.inf, d2)
        iot = jnp.concatenate(
            [lax.broadcasted_iota(jnp.int32, (RB, 128), 1),
             c0 + lax.broadcasted_iota(jnp.int32, (RB, WIN), 1)], axis=1)
        _select_topk(work, iot, idx_ref)

    @pl.when(jnp.logical_not(ok))
    def _():
        xT = xT_ref[...]
        sqc = jnp.sum(xT * xT, axis=0, keepdims=True)
        G = jnp.dot(x16_ref[...], xT16_ref[...],
                    preferred_element_type=jnp.float32)
        d2 = (sqr + sqc) - 2.0 * G
        work = jnp.where(brow1 != bcol_ref[...][:1, :], jnp.inf, d2)
        iot = lax.broadcasted_iota(jnp.int32, (RB, N), 1)
        _select_topk(work, iot, idx_ref)


def _knn(x, x16, brow, bcol, c0s, oks):
    dpad = x.shape[1]
    grid_spec = pltpu.PrefetchScalarGridSpec(
        num_scalar_prefetch=2,
        grid=(N // RB,),
        in_specs=[
            pl.BlockSpec((RB, dpad), lambda i, c, o: (i, 0)),
            pl.BlockSpec((dpad, N), lambda i, c, o: (0, 0)),
            pl.BlockSpec((RB, dpad), lambda i, c, o: (i, 0)),
            pl.BlockSpec((dpad, N), lambda i, c, o: (0, 0)),
            pl.BlockSpec((RB, 8), lambda i, c, o: (i, 0)),
            pl.BlockSpec((8, N), lambda i, c, o: (0, 0)),
        ],
        out_specs=pl.BlockSpec((RB, K), lambda i, c, o: (i, 0)),
    )
    return pl.pallas_call(
        _knn_body,
        grid_spec=grid_spec,
        out_shape=jax.ShapeDtypeStruct((N, K), jnp.int32),
        compiler_params=pltpu.CompilerParams(
            dimension_semantics=("parallel",)),
    )(c0s, oks, x, x.T, x16, x16.T, brow, bcol)


def _gather(x, idxf):
    dpad = x.shape[1]
    rows = idxf.shape[0]
    rpw = rows // NW
    ng = rpw // GRR
    mesh = plsc.VectorSubcoreMesh(core_axis_name="c", subcore_axis_name="s")

    @functools.partial(
        pl.kernel,
        mesh=mesh,
        out_type=jax.ShapeDtypeStruct((rows, dpad), jnp.float32),
        scratch_types=[
            pltpu.VMEM((GRR,), jnp.int32),
            pltpu.VMEM((GRR, dpad), jnp.float32),
            pltpu.SemaphoreType.DMA,
        ],
    )
    def k(x_hbm, idx_hbm, out_hbm, idx_v, rows_v, sem):
        wid = lax.axis_index("s") * 2 + lax.axis_index("c")
        base = wid * rpw

        def group(g, carry):
            nb = base + g * GRR
            pltpu.sync_copy(idx_hbm.at[pl.ds(nb, GRR)], idx_v)
            pltpu.async_copy(x_hbm.at[idx_v], rows_v, sem).wait()
            pltpu.sync_copy(rows_v, out_hbm.at[pl.ds(nb, GRR)])
            return carry

        lax.fori_loop(0, ng, group, 0)

    return k(x, idxf)


def _edge_body(x_ref, xj_ref, W_ref, b_ref, out_ref, *, dpad):
    j = pl.program_id(1)
    xi = x_ref[...]
    xj = xj_ref[...][:, :dpad]
    msg = jnp.concatenate(
        [xi.astype(jnp.bfloat16), (xj - xi).astype(jnp.bfloat16)], axis=1)
    h = jnp.dot(msg, W_ref[...], preferred_element_type=jnp.float32) + b_ref[...]

    @pl.when(j == 0)
    def _():
        out_ref[...] = h

    @pl.when(j > 0)
    def _():
        out_ref[...] = jnp.maximum(out_ref[...], h)


def _edge(x, xj_all, W16, b):
    dpad = x.shape[1]
    gpad = xj_all.shape[1]
    dout = W16.shape[1]
    nb = N // RBM
    return pl.pallas_call(
        functools.partial(_edge_body, dpad=dpad),
        grid=(nb, K),
        in_specs=[
            pl.BlockSpec((RBM, dpad), lambda i, j: (i, 0)),
            pl.BlockSpec((RBM, gpad), lambda i, j: (j * nb + i, 0)),
            pl.BlockSpec((2 * dpad, dout), lambda i, j: (0, 0)),
            pl.BlockSpec((1, dout), lambda i, j: (0, 0)),
        ],
        out_specs=pl.BlockSpec((RBM, dout), lambda i, j: (i, 0)),
        out_shape=jax.ShapeDtypeStruct((N, dout), jnp.float32),
        compiler_params=pltpu.CompilerParams(
            dimension_semantics=("parallel", "arbitrary")),
    )(x, xj_all, W16, b)


def _head_body(x1, x2, x3, x4, brow_ref, bcol_ref, Wm1, bm1, Wa, ba, Wb, bb,
               Wc, bc, out_ref):
    cat = jnp.concatenate([x1[...], x2[...], x3[...], x4[...]], axis=1)
    o = jnp.dot(cat.astype(jnp.bfloat16), Wm1[...],
                preferred_element_type=jnp.float32) + bm1[...]
    bcol = bcol_ref[...][:1, :]
    seg = (lax.broadcasted_iota(jnp.int32, (8, N), 0) == bcol).astype(jnp.float32)
    sums = jnp.dot(seg, o, preferred_element_type=jnp.float32)
    counts = jnp.sum(seg, axis=1, keepdims=True)
    gmean = sums / jnp.maximum(counts, 1.0)
    brow = brow_ref[...][:, :1]
    gmax_rows = []
    for bi in range(8):
        mrow = jnp.max(jnp.where(brow == bi, o, -jnp.inf), axis=0, keepdims=True)
        gmax_rows.append(mrow)
    gmax = jnp.concatenate(gmax_rows, axis=0)
    gmax = jnp.where(counts > 0, gmax, 0.0)
    g = jnp.concatenate([gmax, gmean], axis=1)

    def leaky(v):
        return jnp.where(v >= 0, v, 0.2 * v)

    h = leaky(jnp.dot(g.astype(jnp.bfloat16), Wa[...],
                      preferred_element_type=jnp.float32) + ba[...])
    h = leaky(jnp.dot(h.astype(jnp.bfloat16), Wb[...],
                      preferred_element_type=jnp.float32) + bb[...])
    r = jnp.dot(h.astype(jnp.bfloat16), Wc[...],
                preferred_element_type=jnp.float32) + bc[...]
    out_ref[...] = jnp.pad(r, ((0, 0), (0, 128 - r.shape[1])))


def _head(x1, x2, x3, x4, brow, bcol, Wm1, bm1, Wa, ba, Wb, bb, Wc, bc):
    args = [x1, x2, x3, x4, brow, bcol, Wm1, bm1, Wa, ba, Wb, bb, Wc, bc]

    def full(s):
        return pl.BlockSpec(s, lambda: tuple(0 for _ in s))

    return pl.pallas_call(
        _head_body,
        in_specs=[full(a.shape) for a in args],
        out_specs=full((8, 128)),
        out_shape=jax.ShapeDtypeStruct((8, 128), jnp.float32),
    )(*args)


def kernel(pos, batch, W1, b1, W2, b2, W3, b3, W4, b4, Wm1, bm1, Wa, ba, Wb,
           bb, Wc, bc):
    pos = pos.astype(jnp.float32)
    batch = batch.astype(jnp.int32)
    brow = jnp.broadcast_to(batch[:, None], (N, 8))
    bcol = jnp.broadcast_to(batch[None, :], (8, N))

    x = jnp.pad(pos, ((0, 0), (0, 5)))
    W1p = jnp.concatenate([
        jnp.pad(W1[:3], ((0, 5), (0, 0))),
        jnp.pad(W1[3:], ((0, 5), (0, 0))),
    ], axis=0)

    starts = jnp.searchsorted(batch, jnp.arange(9, dtype=jnp.int32)).astype(jnp.int32)
    bi = jnp.arange(N // RB, dtype=jnp.int32)
    bf = batch[bi * RB]
    bl = batch[bi * RB + RB - 1]
    clo = starts[bf]
    chi = starts[bl + 1]
    c0s = (jnp.minimum(jnp.maximum(clo, 128), N - WIN) // 128) * 128
    oks = (chi - c0s <= WIN).astype(jnp.int32)

    layers = [(W1p, b1), (W2, b2), (W3, b3), (W4, b4)]
    feats = []
    for W, b in layers:
        x16 = x.astype(jnp.bfloat16)
        idx = _knn(x, x16, brow, bcol, c0s, oks)
        idxf = idx.T.reshape(-1)
        dpad = x.shape[1]
        xg = x if dpad >= 128 else jnp.pad(x, ((0, 0), (0, 128 - dpad)))
        xj_all = _gather(xg, idxf)
        x = _edge(x, xj_all, W.astype(jnp.bfloat16), b[None, :])
        feats.append(x)

    out = _head(*feats, brow, bcol, Wm1.astype(jnp.bfloat16), bm1[None, :],
                Wa.astype(jnp.bfloat16), ba[None, :],
                Wb.astype(jnp.bfloat16), bb[None, :],
                Wc.astype(jnp.bfloat16), bc[None, :])
    return out[:, :40]


# speedup vs baseline: 1.1462x; 1.1462x over previous
import functools

import jax
import jax.numpy as jnp
from jax import lax
from jax.experimental import pallas as pl
from jax.experimental.pallas import tpu as pltpu
from jax.experimental.pallas import tpu_sc as plsc

N = 4096
K = 20
RB = 256
RBM = 256
NW = 32
GRR = 64
WIN = 1280


def _select_topk(work, iot, idx_ref):
    BIG = jnp.int32(2 ** 30)
    cols = []
    for _ in range(K):
        m = jnp.min(work, axis=1, keepdims=True)
        cand = jnp.where(work == m, iot, BIG)
        im = jnp.min(cand, axis=1, keepdims=True)
        cols.append(im)
        sel = iot == im
        work = jnp.where(sel, jnp.inf, work)
        iot = jnp.where(sel, BIG, iot)
    idx_ref[...] = jnp.concatenate(cols, axis=1)


def _knn_body(c0_ref, ok_ref, x_ref, xT_ref, x16_ref, xT16_ref, brow_ref,
              bcol_ref, idx_ref):
    i = pl.program_id(0)
    xb = x_ref[...]
    sqr = jnp.sum(xb * xb, axis=1, keepdims=True)
    brow1 = brow_ref[...][:, :1]
    c0 = pl.multiple_of(c0_ref[i], 128)
    ok = ok_ref[i] == 1

    @pl.when(ok)
    def _():
        xTf = jnp.concatenate(
            [xT_ref[:, :128], xT_ref[:, pl.ds(c0, WIN)]], axis=1)
        xTc = jnp.concatenate(
            [xT16_ref[:, :128], xT16_ref[:, pl.ds(c0, WIN)]], axis=1)
        bcolc = jnp.concatenate(
            [bcol_ref[:, :128], bcol_ref[:, pl.ds(c0, WIN)]], axis=1)[:1, :]
        sqc = jnp.sum(xTf * xTf, axis=0, keepdims=True)
        G = jnp.dot(x16_ref[...], xTc, preferred_element_type=jnp.float32)
        d2 = (sqr + sqc) - 2.0 * G
        CW = 128 + WIN
        iotp = lax.broadcasted_iota(jnp.int32, (RB, CW), 1)
        invalid = 1e20 + iotp.astype(jnp.float32) * 1e13
        work = jnp.where(brow1 != bcolc, invalid, d2)
        cols = []
        for _ in range(K):
            im = jnp.argmin(work, axis=1).astype(jnp.int32)[:, None]
            cols.append(jnp.where(im < 128, im, im + (c0 - 128)))
            work = jnp.where(iotp == im, jnp.inf, work)
        idx_ref[...] = jnp.concatenate(cols, axis=1)

    @pl.when(jnp.logical_not(ok))
    def _():
        xT = xT_ref[...]
        sqc = jnp.sum(xT * xT, axis=0, keepdims=True)
        G = jnp.dot(x16_ref[...], xT16_ref[...],
                    preferred_element_type=jnp.float32)
        d2 = (sqr + sqc) - 2.0 * G
        work = jnp.where(brow1 != bcol_ref[...][:1, :], jnp.inf, d2)
        iot = lax.broadcasted_iota(jnp.int32, (RB, N), 1)
        _select_topk(work, iot, idx_ref)


def _knn(x, x16, brow, bcol, c0s, oks):
    dpad = x.shape[1]
    grid_spec = pltpu.PrefetchScalarGridSpec(
        num_scalar_prefetch=2,
        grid=(N // RB,),
        in_specs=[
            pl.BlockSpec((RB, dpad), lambda i, c, o: (i, 0)),
            pl.BlockSpec((dpad, N), lambda i, c, o: (0, 0)),
            pl.BlockSpec((RB, dpad), lambda i, c, o: (i, 0)),
            pl.BlockSpec((dpad, N), lambda i, c, o: (0, 0)),
            pl.BlockSpec((RB, 8), lambda i, c, o: (i, 0)),
            pl.BlockSpec((8, N), lambda i, c, o: (0, 0)),
        ],
        out_specs=pl.BlockSpec((RB, K), lambda i, c, o: (i, 0)),
    )
    return pl.pallas_call(
        _knn_body,
        grid_spec=grid_spec,
        out_shape=jax.ShapeDtypeStruct((N, K), jnp.int32),
        compiler_params=pltpu.CompilerParams(
            dimension_semantics=("parallel",)),
    )(c0s, oks, x, x.T, x16, x16.T, brow, bcol)


def _gather(x, idxf):
    dpad = x.shape[1]
    rows = idxf.shape[0]
    rpw = rows // NW
    ng = rpw // GRR
    mesh = plsc.VectorSubcoreMesh(core_axis_name="c", subcore_axis_name="s")

    @functools.partial(
        pl.kernel,
        mesh=mesh,
        out_type=jax.ShapeDtypeStruct((rows, dpad), jnp.float32),
        scratch_types=[
            pltpu.VMEM((GRR,), jnp.int32),
            pltpu.VMEM((GRR, dpad), jnp.float32),
            pltpu.SemaphoreType.DMA,
        ],
    )
    def k(x_hbm, idx_hbm, out_hbm, idx_v, rows_v, sem):
        wid = lax.axis_index("s") * 2 + lax.axis_index("c")
        base = wid * rpw

        def group(g, carry):
            nb = base + g * GRR
            pltpu.sync_copy(idx_hbm.at[pl.ds(nb, GRR)], idx_v)
            pltpu.async_copy(x_hbm.at[idx_v], rows_v, sem).wait()
            pltpu.sync_copy(rows_v, out_hbm.at[pl.ds(nb, GRR)])
            return carry

        lax.fori_loop(0, ng, group, 0)

    return k(x, idxf)


def _edge_body(x_ref, xj_ref, W_ref, b_ref, out_ref, *, dpad):
    j = pl.program_id(1)
    xi = x_ref[...]
    xj = xj_ref[...][:, :dpad]
    msg = jnp.concatenate(
        [xi.astype(jnp.bfloat16), (xj - xi).astype(jnp.bfloat16)], axis=1)
    h = jnp.dot(msg, W_ref[...], preferred_element_type=jnp.float32) + b_ref[...]

    @pl.when(j == 0)
    def _():
        out_ref[...] = h

    @pl.when(j > 0)
    def _():
        out_ref[...] = jnp.maximum(out_ref[...], h)


def _edge(x, xj_all, W16, b):
    dpad = x.shape[1]
    gpad = xj_all.shape[1]
    dout = W16.shape[1]
    nb = N // RBM
    return pl.pallas_call(
        functools.partial(_edge_body, dpad=dpad),
        grid=(nb, K),
        in_specs=[
            pl.BlockSpec((RBM, dpad), lambda i, j: (i, 0)),
            pl.BlockSpec((RBM, gpad), lambda i, j: (j * nb + i, 0)),
            pl.BlockSpec((2 * dpad, dout), lambda i, j: (0, 0)),
            pl.BlockSpec((1, dout), lambda i, j: (0, 0)),
        ],
        out_specs=pl.BlockSpec((RBM, dout), lambda i, j: (i, 0)),
        out_shape=jax.ShapeDtypeStruct((N, dout), jnp.float32),
        compiler_params=pltpu.CompilerParams(
            dimension_semantics=("parallel", "arbitrary")),
    )(x, xj_all, W16, b)


def _head_body(x1, x2, x3, x4, brow_ref, bcol_ref, Wm1, bm1, Wa, ba, Wb, bb,
               Wc, bc, out_ref):
    cat = jnp.concatenate([x1[...], x2[...], x3[...], x4[...]], axis=1)
    o = jnp.dot(cat.astype(jnp.bfloat16), Wm1[...],
                preferred_element_type=jnp.float32) + bm1[...]
    bcol = bcol_ref[...][:1, :]
    seg = (lax.broadcasted_iota(jnp.int32, (8, N), 0) == bcol).astype(jnp.float32)
    sums = jnp.dot(seg, o, preferred_element_type=jnp.float32)
    counts = jnp.sum(seg, axis=1, keepdims=True)
    gmean = sums / jnp.maximum(counts, 1.0)
    brow = brow_ref[...][:, :1]
    gmax_rows = []
    for bi in range(8):
        mrow = jnp.max(jnp.where(brow == bi, o, -jnp.inf), axis=0, keepdims=True)
        gmax_rows.append(mrow)
    gmax = jnp.concatenate(gmax_rows, axis=0)
    gmax = jnp.where(counts > 0, gmax, 0.0)
    g = jnp.concatenate([gmax, gmean], axis=1)

    def leaky(v):
        return jnp.where(v >= 0, v, 0.2 * v)

    h = leaky(jnp.dot(g.astype(jnp.bfloat16), Wa[...],
                      preferred_element_type=jnp.float32) + ba[...])
    h = leaky(jnp.dot(h.astype(jnp.bfloat16), Wb[...],
                      preferred_element_type=jnp.float32) + bb[...])
    r = jnp.dot(h.astype(jnp.bfloat16), Wc[...],
                preferred_element_type=jnp.float32) + bc[...]
    out_ref[...] = jnp.pad(r, ((0, 0), (0, 128 - r.shape[1])))


def _head(x1, x2, x3, x4, brow, bcol, Wm1, bm1, Wa, ba, Wb, bb, Wc, bc):
    args = [x1, x2, x3, x4, brow, bcol, Wm1, bm1, Wa, ba, Wb, bb, Wc, bc]

    def full(s):
        return pl.BlockSpec(s, lambda: tuple(0 for _ in s))

    return pl.pallas_call(
        _head_body,
        in_specs=[full(a.shape) for a in args],
        out_specs=full((8, 128)),
        out_shape=jax.ShapeDtypeStruct((8, 128), jnp.float32),
    )(*args)


def kernel(pos, batch, W1, b1, W2, b2, W3, b3, W4, b4, Wm1, bm1, Wa, ba, Wb,
           bb, Wc, bc):
    pos = pos.astype(jnp.float32)
    batch = batch.astype(jnp.int32)
    brow = jnp.broadcast_to(batch[:, None], (N, 8))
    bcol = jnp.broadcast_to(batch[None, :], (8, N))

    x = jnp.pad(pos, ((0, 0), (0, 5)))
    W1p = jnp.concatenate([
        jnp.pad(W1[:3], ((0, 5), (0, 0))),
        jnp.pad(W1[3:], ((0, 5), (0, 0))),
    ], axis=0)

    starts = jnp.searchsorted(batch, jnp.arange(9, dtype=jnp.int32)).astype(jnp.int32)
    bi = jnp.arange(N // RB, dtype=jnp.int32)
    bf = batch[bi * RB]
    bl = batch[bi * RB + RB - 1]
    clo = starts[bf]
    chi = starts[bl + 1]
    c0s = (jnp.minimum(jnp.maximum(clo, 128), N - WIN) // 128) * 128
    oks = (chi - c0s <= WIN).astype(jnp.int32)

    layers = [(W1p, b1), (W2, b2), (W3, b3), (W4, b4)]
    feats = []
    for W, b in layers:
        x16 = x.astype(jnp.bfloat16)
        idx = _knn(x, x16, brow, bcol, c0s, oks)
        idxf = idx.T.reshape(-1)
        dpad = x.shape[1]
        xg = x if dpad >= 128 else jnp.pad(x, ((0, 0), (0, 128 - dpad)))
        xj_all = _gather(xg, idxf)
        x = _edge(x, xj_all, W.astype(jnp.bfloat16), b[None, :])
        feats.append(x)

    out = _head(*feats, brow, bcol, Wm1.astype(jnp.bfloat16), bm1[None, :],
                Wa.astype(jnp.bfloat16), ba[None, :],
                Wb.astype(jnp.bfloat16), bb[None, :],
                Wc.astype(jnp.bfloat16), bc[None, :])
    return out[:, :40]


# speedup vs baseline: 2.0647x; 1.8014x over previous
import functools

import jax
import jax.numpy as jnp
from jax import lax
from jax.experimental import pallas as pl
from jax.experimental.pallas import tpu as pltpu
from jax.experimental.pallas import tpu_sc as plsc

N = 4096
K = 20
RB = 256
RBM = 256
NW = 32
GRR = 64
WIN = 1280


def _select_topk(work, iot, idx_ref):
    BIG = jnp.int32(2 ** 30)
    cols = []
    for _ in range(K):
        m = jnp.min(work, axis=1, keepdims=True)
        cand = jnp.where(work == m, iot, BIG)
        im = jnp.min(cand, axis=1, keepdims=True)
        cols.append(im)
        sel = iot == im
        work = jnp.where(sel, jnp.inf, work)
        iot = jnp.where(sel, BIG, iot)
    idx_ref[...] = jnp.concatenate(cols, axis=1)


def _knn_body(c0_ref, ok_ref, x_ref, xT_ref, x16_ref, xT16_ref, brow_ref,
              bcol_ref, idx_ref):
    i = pl.program_id(0)
    xb = x_ref[...]
    sqr = jnp.sum(xb * xb, axis=1, keepdims=True)
    brow1 = brow_ref[...][:, :1]
    c0 = pl.multiple_of(c0_ref[i], 128)
    ok = ok_ref[i] == 1

    @pl.when(ok)
    def _():
        xTf = jnp.concatenate(
            [xT_ref[:, :128], xT_ref[:, pl.ds(c0, WIN)]], axis=1)
        xTc = jnp.concatenate(
            [xT16_ref[:, :128], xT16_ref[:, pl.ds(c0, WIN)]], axis=1)
        bcolc = jnp.concatenate(
            [bcol_ref[:, :128], bcol_ref[:, pl.ds(c0, WIN)]], axis=1)[:1, :]
        sqc = jnp.sum(xTf * xTf, axis=0, keepdims=True)
        G = jnp.dot(x16_ref[...], xTc, preferred_element_type=jnp.float32)
        d2 = (sqr + sqc) - 2.0 * G
        CW = 128 + WIN
        iotp = lax.broadcasted_iota(jnp.int32, (RB, CW), 1)
        invalid = 1e20 + iotp.astype(jnp.float32) * 1e13
        work = jnp.where(brow1 != bcolc, invalid, d2)
        cols = []
        for _ in range(K):
            im = jnp.argmin(work, axis=1).astype(jnp.int32)[:, None]
            cols.append(jnp.where(im < 128, im, im + (c0 - 128)))
            work = jnp.where(iotp == im, jnp.inf, work)
        idx_ref[...] = jnp.concatenate(cols, axis=1)

    @pl.when(jnp.logical_not(ok))
    def _():
        xT = xT_ref[...]
        sqc = jnp.sum(xT * xT, axis=0, keepdims=True)
        G = jnp.dot(x16_ref[...], xT16_ref[...],
                    preferred_element_type=jnp.float32)
        d2 = (sqr + sqc) - 2.0 * G
        work = jnp.where(brow1 != bcol_ref[...][:1, :], jnp.inf, d2)
        iot = lax.broadcasted_iota(jnp.int32, (RB, N), 1)
        _select_topk(work, iot, idx_ref)


def _knn(x, x16, brow, bcol, c0s, oks):
    dpad = x.shape[1]
    grid_spec = pltpu.PrefetchScalarGridSpec(
        num_scalar_prefetch=2,
        grid=(N // RB,),
        in_specs=[
            pl.BlockSpec((RB, dpad), lambda i, c, o: (i, 0)),
            pl.BlockSpec((dpad, N), lambda i, c, o: (0, 0)),
            pl.BlockSpec((RB, dpad), lambda i, c, o: (i, 0)),
            pl.BlockSpec((dpad, N), lambda i, c, o: (0, 0)),
            pl.BlockSpec((RB, 8), lambda i, c, o: (i, 0)),
            pl.BlockSpec((8, N), lambda i, c, o: (0, 0)),
        ],
        out_specs=pl.BlockSpec((RB, K), lambda i, c, o: (i, 0)),
    )
    return pl.pallas_call(
        _knn_body,
        grid_spec=grid_spec,
        out_shape=jax.ShapeDtypeStruct((N, K), jnp.int32),
        compiler_params=pltpu.CompilerParams(
            dimension_semantics=("parallel",)),
    )(c0s, oks, x, x.T, x16, x16.T, brow, bcol)


def _gather(x, idxf):
    dpad = x.shape[1]
    rows = idxf.shape[0]
    rpw = rows // NW
    grr = GRR if dpad > 128 else 2 * GRR
    ng = rpw // grr
    mesh = plsc.VectorSubcoreMesh(core_axis_name="c", subcore_axis_name="s")

    @functools.partial(
        pl.kernel,
        mesh=mesh,
        out_type=jax.ShapeDtypeStruct((rows, dpad), jnp.float32),
        scratch_types=[
            pltpu.VMEM((grr,), jnp.int32),
            pltpu.VMEM((grr, dpad), jnp.float32),
            pltpu.SemaphoreType.DMA,
        ],
    )
    def k(x_hbm, idx_hbm, out_hbm, idx_v, rows_v, sem):
        wid = lax.axis_index("s") * 2 + lax.axis_index("c")
        base = wid * rpw

        def group(g, carry):
            nb = base + g * grr
            pltpu.sync_copy(idx_hbm.at[pl.ds(nb, grr)], idx_v)
            pltpu.async_copy(x_hbm.at[idx_v], rows_v, sem).wait()
            pltpu.sync_copy(rows_v, out_hbm.at[pl.ds(nb, grr)])
            return carry

        lax.fori_loop(0, ng, group, 0)

    return k(x, idxf)


def _edge_body(x_ref, xj_ref, W_ref, b_ref, out_ref, *, dpad):
    xi = x_ref[...]
    xi16 = xi.astype(jnp.bfloat16)
    W = W_ref[...]
    b = b_ref[...]
    out = None
    for j in range(K):
        xj = xj_ref[j][:, :dpad]
        msg = jnp.concatenate([xi16, (xj - xi).astype(jnp.bfloat16)], axis=1)
        h = jnp.dot(msg, W, preferred_element_type=jnp.float32) + b
        out = h if out is None else jnp.maximum(out, h)
    out_ref[...] = out


def _edge(x, xj_all, W16, b):
    dpad = x.shape[1]
    gpad = xj_all.shape[2]
    dout = W16.shape[1]
    nb = N // RBM
    return pl.pallas_call(
        functools.partial(_edge_body, dpad=dpad),
        grid=(nb,),
        in_specs=[
            pl.BlockSpec((RBM, dpad), lambda i: (i, 0)),
            pl.BlockSpec((K, RBM, gpad), lambda i: (0, i, 0)),
            pl.BlockSpec((2 * dpad, dout), lambda i: (0, 0)),
            pl.BlockSpec((1, dout), lambda i: (0, 0)),
        ],
        out_specs=pl.BlockSpec((RBM, dout), lambda i: (i, 0)),
        out_shape=jax.ShapeDtypeStruct((N, dout), jnp.float32),
        compiler_params=pltpu.CompilerParams(
            dimension_semantics=("parallel",)),
    )(x, xj_all, W16, b)


def _head_body(x1, x2, x3, x4, brow_ref, bcol_ref, Wm1, bm1, Wa, ba, Wb, bb,
               Wc, bc, out_ref):
    cat = jnp.concatenate([x1[...], x2[...], x3[...], x4[...]], axis=1)
    o = jnp.dot(cat.astype(jnp.bfloat16), Wm1[...],
                preferred_element_type=jnp.float32) + bm1[...]
    bcol = bcol_ref[...][:1, :]
    seg = (lax.broadcasted_iota(jnp.int32, (8, N), 0) == bcol).astype(jnp.float32)
    sums = jnp.dot(seg, o, preferred_element_type=jnp.float32)
    counts = jnp.sum(seg, axis=1, keepdims=True)
    gmean = sums / jnp.maximum(counts, 1.0)
    brow = brow_ref[...][:, :1]
    gmax_rows = []
    for bi in range(8):
        mrow = jnp.max(jnp.where(brow == bi, o, -jnp.inf), axis=0, keepdims=True)
        gmax_rows.append(mrow)
    gmax = jnp.concatenate(gmax_rows, axis=0)
    gmax = jnp.where(counts > 0, gmax, 0.0)
    g = jnp.concatenate([gmax, gmean], axis=1)

    def leaky(v):
        return jnp.where(v >= 0, v, 0.2 * v)

    h = leaky(jnp.dot(g.astype(jnp.bfloat16), Wa[...],
                      preferred_element_type=jnp.float32) + ba[...])
    h = leaky(jnp.dot(h.astype(jnp.bfloat16), Wb[...],
                      preferred_element_type=jnp.float32) + bb[...])
    r = jnp.dot(h.astype(jnp.bfloat16), Wc[...],
                preferred_element_type=jnp.float32) + bc[...]
    out_ref[...] = jnp.pad(r, ((0, 0), (0, 128 - r.shape[1])))


def _head(x1, x2, x3, x4, brow, bcol, Wm1, bm1, Wa, ba, Wb, bb, Wc, bc):
    args = [x1, x2, x3, x4, brow, bcol, Wm1, bm1, Wa, ba, Wb, bb, Wc, bc]

    def full(s):
        return pl.BlockSpec(s, lambda: tuple(0 for _ in s))

    return pl.pallas_call(
        _head_body,
        in_specs=[full(a.shape) for a in args],
        out_specs=full((8, 128)),
        out_shape=jax.ShapeDtypeStruct((8, 128), jnp.float32),
    )(*args)


def kernel(pos, batch, W1, b1, W2, b2, W3, b3, W4, b4, Wm1, bm1, Wa, ba, Wb,
           bb, Wc, bc):
    pos = pos.astype(jnp.float32)
    batch = batch.astype(jnp.int32)
    brow = jnp.broadcast_to(batch[:, None], (N, 8))
    bcol = jnp.broadcast_to(batch[None, :], (8, N))

    x = jnp.pad(pos, ((0, 0), (0, 5)))
    W1p = jnp.concatenate([
        jnp.pad(W1[:3], ((0, 5), (0, 0))),
        jnp.pad(W1[3:], ((0, 5), (0, 0))),
    ], axis=0)

    starts = jnp.searchsorted(batch, jnp.arange(9, dtype=jnp.int32)).astype(jnp.int32)
    bi = jnp.arange(N // RB, dtype=jnp.int32)
    bf = batch[bi * RB]
    bl = batch[bi * RB + RB - 1]
    clo = starts[bf]
    chi = starts[bl + 1]
    c0s = (jnp.minimum(jnp.maximum(clo, 128), N - WIN) // 128) * 128
    oks = (chi - c0s <= WIN).astype(jnp.int32)

    layers = [(W1p, b1), (W2, b2), (W3, b3), (W4, b4)]
    feats = []
    for W, b in layers:
        x16 = x.astype(jnp.bfloat16)
        idx = _knn(x, x16, brow, bcol, c0s, oks)
        idxf = idx.T.reshape(-1)
        dpad = x.shape[1]
        xg = x if dpad >= 128 else jnp.pad(x, ((0, 0), (0, 128 - dpad)))
        gpad = xg.shape[1]
        xj_all = _gather(xg, idxf).reshape(K, N, gpad)
        x = _edge(x, xj_all, W.astype(jnp.bfloat16), b[None, :])
        feats.append(x)

    out = _head(*feats, brow, bcol, Wm1.astype(jnp.bfloat16), bm1[None, :],
                Wa.astype(jnp.bfloat16), ba[None, :],
                Wb.astype(jnp.bfloat16), bb[None, :],
                Wc.astype(jnp.bfloat16), bc[None, :])
    return out[:, :40]
